# baseline (speedup 1.0000x reference)
import functools

import jax
import jax.numpy as jnp
from jax import lax
from jax.experimental import pallas as pl
from jax.experimental.pallas import tpu as pltpu
from jax.experimental.pallas import tpu_sc as plsc

_B, _C, _N, _K = 4, 32, 4096, 24
_OC3 = 3 * _C
_E = _B * _N * _K
_EPB = _N * _K
_RB = 256
_MB = 2048
_NB = 256
_NW = 32
_CH = 128
_NCHUNK = _E // (_NW * _CH)


_NP = 32
_PW = _N // _NP
_T = 8

def _knn_scores(xvt_ref, xv_ref):
    xr = xvt_ref[0]
    xv = xv_ref[0]
    dot = lax.dot_general(xr, xv, (((1,), (0,)), ((), ())),
                          preferred_element_type=jnp.float32)
    inner = -2.0 * dot
    xx_r = jnp.sum(xr * xr, axis=1, keepdims=True)
    xx_c = jnp.sum(xv * xv, axis=0, keepdims=True)
    return -xx_r - inner - xx_c


def _knn_body(xvt_ref, xv_ref, out_ref):
    b = pl.program_id(0)
    neg = jnp.float32(-3.0e38)
    big = jnp.int32(1 << 30)
    s = _knn_scores(xvt_ref, xv_ref)
    tcol = lax.broadcasted_iota(jnp.int32, (_RB, _K), 1)

    colp = lax.broadcasted_iota(jnp.int32, (_RB, _PW), 1)
    vals, cols = [], []
    for p in range(_NP):
        sp = s[:, p * _PW:(p + 1) * _PW]
        for i in range(_T):
            m = jnp.max(sp, axis=1, keepdims=True)
            cand = jnp.where(sp == m, colp, big)
            am = jnp.min(cand, axis=1, keepdims=True)
            vals.append(m)
            cols.append(am + p * _PW)
            sp = jnp.where(colp == am, neg, sp)
    v = jnp.concatenate(vals, axis=1)
    ci = jnp.concatenate(cols, axis=1)
    v8 = jnp.concatenate([vals[p * _T + _T - 1] for p in range(_NP)], axis=1)

    acc = jnp.zeros((_RB, _K), jnp.int32)
    v24 = None
    for t in range(_K):
        m = jnp.max(v, axis=1, keepdims=True)
        cand = jnp.where(v == m, ci, big)
        am = jnp.min(cand, axis=1, keepdims=True)
        acc = jnp.where(tcol == t, jnp.broadcast_to(am, (_RB, _K)), acc)
        v = jnp.where((v == m) & (ci == am), neg, v)
        if t == _K - 1:
            v24 = m
    out_ref[0] = acc + b * _N

    bad = jnp.max(jnp.where(v8 >= v24, 1, 0))

    @pl.when(bad > 0)
    def _():
        sf = _knn_scores(xvt_ref, xv_ref)
        colf = lax.broadcasted_iota(jnp.int32, (_RB, _N), 1)
        accf = jnp.zeros((_RB, _K), jnp.int32)
        for t in range(_K):
            m = jnp.max(sf, axis=1, keepdims=True)
            cand = jnp.where(sf == m, colf, big)
            am = jnp.min(cand, axis=1, keepdims=True)
            accf = jnp.where(tcol == t, jnp.broadcast_to(am, (_RB, _K)),
                             accf)
            sf = jnp.where(colf == am, neg, sf)
        out_ref[0] = accf + b * _N


def _knn(xvt, xv):
    return pl.pallas_call(
        _knn_body,
        grid=(_B, _N // _RB),
        in_specs=[
            pl.BlockSpec((1, _RB, _C), lambda b, r: (b, r, 0)),
            pl.BlockSpec((1, _C, _N), lambda b, r: (b, 0, 0)),
        ],
        out_specs=pl.BlockSpec((1, _RB, _K), lambda b, r: (b, r, 0)),
        out_shape=jax.ShapeDtypeStruct((_B, _N, _K), jnp.int32),
    )(xvt, xv)


def _proj_body(xc_ref, xn_ref, wac_ref, wan_ref, wgc_ref, wgn_ref, bias_ref,
               a_ref, g_ref):
    xc = xc_ref[0]
    xn = xn_ref[0]
    dn = lambda x, w: lax.dot_general(x, w, (((1,), (0,)), ((), ())),
                                      preferred_element_type=jnp.float32)
    a = jnp.concatenate([dn(xc, wac_ref[...]), dn(xn, wan_ref[...])], axis=1)
    a_ref[0] = a + bias_ref[0:1, :]
    g_ref[0] = jnp.concatenate([dn(xc, wgc_ref[...]), dn(xn, wgn_ref[...])],
                               axis=1)


def _proj(xc_p, xn_p, wac, wan, wgc, wgn, bias):
    full = lambda shape: pl.BlockSpec(shape, lambda b: (0,) * len(shape))
    return pl.pallas_call(
        _proj_body,
        grid=(_B,),
        in_specs=[
            pl.BlockSpec((1, _N, _C), lambda b: (b, 0, 0)),
            pl.BlockSpec((1, _N, _C), lambda b: (b, 0, 0)),
            full((_C, _C)), full((_C, 2 * _C)),
            full((_C, _C)), full((_C, 2 * _C)),
            full((8, _OC3)),
        ],
        out_specs=[
            pl.BlockSpec((1, _N, _OC3), lambda b: (b, 0, 0)),
            pl.BlockSpec((1, _N, _OC3), lambda b: (b, 0, 0)),
        ],
        out_shape=[
            jax.ShapeDtypeStruct((_B, _N, _OC3), jnp.float32),
            jax.ShapeDtypeStruct((_B, _N, _OC3), jnp.float32),
        ],
    )(xc_p, xn_p, wac, wan, wgc, wgn, bias)


def _gather_body(table_ref, idx_ref, out_ref, idx_v, rows_v, sem):
    wid = lax.axis_index("s") * 2 + lax.axis_index("c")

    def chunk(c, carry):
        pltpu.sync_copy(idx_ref.at[wid, c], idx_v)
        pltpu.async_copy(table_ref.at[idx_v], rows_v, sem).wait()
        base = (wid * _NCHUNK + c) * _CH
        pltpu.sync_copy(rows_v, out_ref.at[pl.ds(base, _CH)])
        return carry

    lax.fori_loop(0, _NCHUNK, chunk, 0)


def _sc_gather(table, idx):
    mesh = plsc.VectorSubcoreMesh(core_axis_name="c", subcore_axis_name="s")
    f = functools.partial(
        pl.kernel,
        mesh=mesh,
        compiler_params=pltpu.CompilerParams(use_tc_tiling_on_sc=False),
        out_type=jax.ShapeDtypeStruct((_E, _OC3), jnp.float32),
        scratch_types=[
            pltpu.VMEM((_CH,), jnp.int32),
            pltpu.VMEM((_CH, _OC3), jnp.float32),
            pltpu.SemaphoreType.DMA,
        ],
    )(_gather_body)
    return f(table, idx)


def _stats_body(a_ref, h_ref, gb_ref, o_ref):
    b = pl.program_id(0)
    pb = pl.program_id(1)
    y = a_ref[0] + h_ref[0]
    ps = jnp.sum(y, axis=0, keepdims=True)
    pq = jnp.sum(y * y, axis=0, keepdims=True)

    @pl.when((b == 0) & (pb == 0))
    def _():
        o_ref[...] = jnp.zeros((8, _OC3), jnp.float32)

    o_ref[0:1, :] += ps
    o_ref[1:2, :] += pq

    @pl.when((b == _B - 1) & (pb == _EPB // _MB - 1))
    def _():
        inv_e = jnp.float32(1.0 / _E)
        mean = o_ref[0:1, :] * inv_e
        var = o_ref[1:2, :] * inv_e - mean * mean
        scale = gb_ref[0:1, :] / jnp.sqrt(var + 1e-5)
        shift = gb_ref[1:2, :] - mean * scale
        o_ref[0:1, :] = scale
        o_ref[1:2, :] = shift


def _stats(a_tab, h_edges, gb):
    return pl.pallas_call(
        _stats_body,
        grid=(_B, _EPB // _MB),
        in_specs=[
            pl.BlockSpec((1, _MB, _OC3),
                         lambda b, p: (b, p % (_N // _MB), 0)),
            pl.BlockSpec((1, _MB, _OC3), lambda b, p: (b, p, 0)),
            pl.BlockSpec((2, _OC3), lambda b, p: (0, 0)),
        ],
        out_specs=pl.BlockSpec((8, _OC3), lambda b, p: (0, 0)),
        out_shape=jax.ShapeDtypeStruct((8, _OC3), jnp.float32),
    )(a_tab, h_edges, gb)


def _edge_body(a_ref, h_ref, st_ref, xc_ref, xn_ref):
    scale = st_ref[0:1, :]
    shift = st_ref[1:2, :]
    a = a_ref[0]
    zts = []
    for j in range(_K):
        y = a + h_ref[0, j]
        z = y * scale + shift
        z = jnp.where(z >= 0, z, 0.01 * z)
        zts.append(jnp.transpose(z, (1, 0)))
    z3 = jnp.stack(zts, axis=0)

    def group(br, t):
        u0 = 32 * t
        c_lo, j0 = u0 // _K, u0 % _K
        c0 = br * _C + c_lo
        p1 = z3[j0:_K, c0, :]
        p2 = z3[0:32 - (_K - j0), c0 + 1, :]
        return jnp.concatenate([p1, p2], axis=0)

    acc = jnp.zeros((_C, _NB), jnp.float32)
    mx = None
    for t in range(_K):
        g1 = group(1, t)
        e = jnp.exp(g1 - jnp.max(g1, axis=0, keepdims=True))
        attn = e / jnp.sum(e, axis=0, keepdims=True)
        acc = acc + attn * group(0, t)
        g2 = group(2, t)
        mx = g2 if t == 0 else jnp.maximum(mx, g2)
    xc_ref[0] = acc
    xn_ref[0] = mx


def _edge(a_tab, h4, st):
    return pl.pallas_call(
        _edge_body,
        grid=(_B, _N // _NB),
        in_specs=[
            pl.BlockSpec((1, _NB, _OC3), lambda b, n: (b, n, 0)),
            pl.BlockSpec((1, _K, _NB, _OC3), lambda b, n: (b, 0, n, 0)),
            pl.BlockSpec((8, _OC3), lambda b, n: (0, 0)),
        ],
        out_specs=[
            pl.BlockSpec((1, _C, _NB), lambda b, n: (b, 0, n)),
            pl.BlockSpec((1, _C, _NB), lambda b, n: (b, 0, n)),
        ],
        out_shape=[
            jax.ShapeDtypeStruct((_B, _C, _N), jnp.float32),
            jax.ShapeDtypeStruct((_B, _C, _N), jnp.float32),
        ],
    )(a_tab, h4, st)


def kernel(xc, xn, W0, b0, g0, be0, W1, b1, g1, be1, W2, b2, g2, be2):
    xc_p = jnp.transpose(xc, (0, 2, 1))
    xn_p = jnp.transpose(xn, (0, 2, 1))
    xv = xc_p.reshape(_B, _C, _N)
    xvt = jnp.transpose(xv, (0, 2, 1))

    idx = _knn(xvt, xv)

    wa0, wg0 = W0[:, :_C], W0[:, _C:]
    wa1, wg1 = W1[:, :_C], W1[:, _C:]
    wa2, wg2 = W2[:, :_C], W2[:, _C:]
    wac = wa0.T
    wan = jnp.concatenate([(wa1 + wg1).T, wa2.T], axis=1)
    wgc = wg0.T
    wgn = jnp.concatenate([(-wg1).T, wg2.T], axis=1)
    bias = jnp.broadcast_to(
        jnp.concatenate([b0, b1, b2])[None, :], (8, _OC3))
    a_tab, g_tab = _proj(xc_p, xn_p, wac, wan, wgc, wgn, bias)

    h = _sc_gather(g_tab.reshape(_B * _N, _OC3),
                   idx.reshape(_NW, _NCHUNK, _CH))

    gb = jnp.stack([jnp.concatenate([g0, g1, g2]),
                    jnp.concatenate([be0, be1, be2])])
    st = _stats(a_tab, h.reshape(_B, _EPB, _OC3), gb)

    return _edge(a_tab, h.reshape(_B, _K, _N, _OC3), st)


# speedup vs baseline: 2.5811x; 2.5811x over previous
import functools

import jax
import jax.numpy as jnp
from jax import lax
from jax.experimental import pallas as pl
from jax.experimental.pallas import tpu as pltpu
from jax.experimental.pallas import tpu_sc as plsc

_B, _C, _N, _K = 4, 32, 4096, 24
_OC3 = 3 * _C
_E = _B * _N * _K
_EPB = _N * _K
_RB = 256
_MB = 2048
_NB = 256
_NW = 32
_CH = 128
_NCHUNK = _E // (_NW * _CH)


def _knn_body(xvt_ref, xv_ref, out_ref):
    b = pl.program_id(0)
    xr = xvt_ref[0]
    xv = xv_ref[0]
    dot = lax.dot_general(xr, xv, (((1,), (0,)), ((), ())),
                          preferred_element_type=jnp.float32)
    inner = -2.0 * dot
    xx_r = jnp.sum(xr * xr, axis=1, keepdims=True)
    xx_c = jnp.sum(xv * xv, axis=0, keepdims=True)
    s = -xx_r - inner - xx_c
    col = lax.broadcasted_iota(jnp.int32, (_RB, _N), 1).astype(jnp.float32)
    tcol = lax.broadcasted_iota(jnp.int32, (_RB, _K), 1).astype(jnp.float32)
    acc = jnp.zeros((_RB, _K), jnp.float32)
    big = jnp.float32(3.0e38)
    for t in range(_K):
        m = jnp.max(s, axis=1, keepdims=True)
        cand = jnp.where(s == m, col, big)
        am = jnp.min(cand, axis=1, keepdims=True)
        acc = jnp.where(tcol == t, jnp.broadcast_to(am, (_RB, _K)), acc)
        s = jnp.where(col == am, jnp.float32(-3.0e38), s)
    out_ref[0] = acc.astype(jnp.int32) + b * _N


def _knn(xvt, xv):
    return pl.pallas_call(
        _knn_body,
        grid=(_B, _N // _RB),
        in_specs=[
            pl.BlockSpec((1, _RB, _C), lambda b, r: (b, r, 0)),
            pl.BlockSpec((1, _C, _N), lambda b, r: (b, 0, 0)),
        ],
        out_specs=pl.BlockSpec((1, _RB, _K), lambda b, r: (b, r, 0)),
        out_shape=jax.ShapeDtypeStruct((_B, _N, _K), jnp.int32),
    )(xvt, xv)


def _proj_body(xc_ref, xn_ref, wac_ref, wan_ref, wgc_ref, wgn_ref, bias_ref,
               a_ref, g_ref):
    xc = xc_ref[0]
    xn = xn_ref[0]
    dn = lambda x, w: lax.dot_general(x, w, (((1,), (0,)), ((), ())),
                                      preferred_element_type=jnp.float32)
    a = jnp.concatenate([dn(xc, wac_ref[...]), dn(xn, wan_ref[...])], axis=1)
    a_ref[0] = a + bias_ref[0:1, :]
    g_ref[0] = jnp.concatenate([dn(xc, wgc_ref[...]), dn(xn, wgn_ref[...])],
                               axis=1)


def _proj(xc_p, xn_p, wac, wan, wgc, wgn, bias):
    full = lambda shape: pl.BlockSpec(shape, lambda b: (0,) * len(shape))
    return pl.pallas_call(
        _proj_body,
        grid=(_B,),
        in_specs=[
            pl.BlockSpec((1, _N, _C), lambda b: (b, 0, 0)),
            pl.BlockSpec((1, _N, _C), lambda b: (b, 0, 0)),
            full((_C, _C)), full((_C, 2 * _C)),
            full((_C, _C)), full((_C, 2 * _C)),
            full((8, _OC3)),
        ],
        out_specs=[
            pl.BlockSpec((1, _N, _OC3), lambda b: (b, 0, 0)),
            pl.BlockSpec((1, _N, _OC3), lambda b: (b, 0, 0)),
        ],
        out_shape=[
            jax.ShapeDtypeStruct((_B, _N, _OC3), jnp.float32),
            jax.ShapeDtypeStruct((_B, _N, _OC3), jnp.float32),
        ],
    )(xc_p, xn_p, wac, wan, wgc, wgn, bias)


def _gather_body(table_ref, idx_ref, out_ref, idx_v, rows_v, sem):
    wid = lax.axis_index("s") * 2 + lax.axis_index("c")

    def chunk(c, carry):
        pltpu.sync_copy(idx_ref.at[wid, c], idx_v)
        pltpu.async_copy(table_ref.at[idx_v], rows_v, sem).wait()
        base = (wid * _NCHUNK + c) * _CH
        pltpu.sync_copy(rows_v, out_ref.at[pl.ds(base, _CH)])
        return carry

    lax.fori_loop(0, _NCHUNK, chunk, 0)


def _sc_gather(table, idx):
    mesh = plsc.VectorSubcoreMesh(core_axis_name="c", subcore_axis_name="s")
    f = functools.partial(
        pl.kernel,
        mesh=mesh,
        compiler_params=pltpu.CompilerParams(use_tc_tiling_on_sc=False),
        out_type=jax.ShapeDtypeStruct((_E, _OC3), jnp.float32),
        scratch_types=[
            pltpu.VMEM((_CH,), jnp.int32),
            pltpu.VMEM((_CH, _OC3), jnp.float32),
            pltpu.SemaphoreType.DMA,
        ],
    )(_gather_body)
    return f(table, idx)


def _stats_body(a_ref, h_ref, gb_ref, o_ref):
    b = pl.program_id(0)
    pb = pl.program_id(1)
    y = a_ref[0] + h_ref[0]
    ps = jnp.sum(y, axis=0, keepdims=True)
    pq = jnp.sum(y * y, axis=0, keepdims=True)

    @pl.when((b == 0) & (pb == 0))
    def _():
        o_ref[...] = jnp.zeros((8, _OC3), jnp.float32)

    o_ref[0:1, :] += ps
    o_ref[1:2, :] += pq

    @pl.when((b == _B - 1) & (pb == _EPB // _MB - 1))
    def _():
        inv_e = jnp.float32(1.0 / _E)
        mean = o_ref[0:1, :] * inv_e
        var = o_ref[1:2, :] * inv_e - mean * mean
        scale = gb_ref[0:1, :] / jnp.sqrt(var + 1e-5)
        shift = gb_ref[1:2, :] - mean * scale
        o_ref[0:1, :] = scale
        o_ref[1:2, :] = shift


def _stats(a_tab, h_edges, gb):
    return pl.pallas_call(
        _stats_body,
        grid=(_B, _EPB // _MB),
        in_specs=[
            pl.BlockSpec((1, _MB, _OC3),
                         lambda b, p: (b, p % (_N // _MB), 0)),
            pl.BlockSpec((1, _MB, _OC3), lambda b, p: (b, p, 0)),
            pl.BlockSpec((2, _OC3), lambda b, p: (0, 0)),
        ],
        out_specs=pl.BlockSpec((8, _OC3), lambda b, p: (0, 0)),
        out_shape=jax.ShapeDtypeStruct((8, _OC3), jnp.float32),
    )(a_tab, h_edges, gb)


def _edge_body(a_ref, h_ref, st_ref, xc_ref, xn_ref):
    scale = st_ref[0:1, :]
    shift = st_ref[1:2, :]
    a = a_ref[0]
    zts = []
    for j in range(_K):
        y = a + h_ref[0, j]
        z = y * scale + shift
        z = jnp.where(z >= 0, z, 0.01 * z)
        zts.append(jnp.transpose(z, (1, 0)))
    z3 = jnp.stack(zts, axis=0)

    def group(br, t):
        u0 = 32 * t
        c_lo, j0 = u0 // _K, u0 % _K
        c0 = br * _C + c_lo
        p1 = z3[j0:_K, c0, :]
        p2 = z3[0:32 - (_K - j0), c0 + 1, :]
        return jnp.concatenate([p1, p2], axis=0)

    acc = jnp.zeros((_C, _NB), jnp.float32)
    mx = None
    for t in range(_K):
        g1 = group(1, t)
        e = jnp.exp(g1 - jnp.max(g1, axis=0, keepdims=True))
        attn = e / jnp.sum(e, axis=0, keepdims=True)
        acc = acc + attn * group(0, t)
        g2 = group(2, t)
        mx = g2 if t == 0 else jnp.maximum(mx, g2)
    xc_ref[0] = acc
    xn_ref[0] = mx


def _edge(a_tab, h4, st):
    return pl.pallas_call(
        _edge_body,
        grid=(_B, _N // _NB),
        in_specs=[
            pl.BlockSpec((1, _NB, _OC3), lambda b, n: (b, n, 0)),
            pl.BlockSpec((1, _K, _NB, _OC3), lambda b, n: (b, 0, n, 0)),
            pl.BlockSpec((8, _OC3), lambda b, n: (0, 0)),
        ],
        out_specs=[
            pl.BlockSpec((1, _C, _NB), lambda b, n: (b, 0, n)),
            pl.BlockSpec((1, _C, _NB), lambda b, n: (b, 0, n)),
        ],
        out_shape=[
            jax.ShapeDtypeStruct((_B, _C, _N), jnp.float32),
            jax.ShapeDtypeStruct((_B, _C, _N), jnp.float32),
        ],
    )(a_tab, h4, st)


def kernel(xc, xn, W0, b0, g0, be0, W1, b1, g1, be1, W2, b2, g2, be2):
    xc_p = jnp.transpose(xc, (0, 2, 1))
    xn_p = jnp.transpose(xn, (0, 2, 1))
    xv = xc_p.reshape(_B, _C, _N)
    xvt = jnp.transpose(xv, (0, 2, 1))

    idx = _knn(xvt, xv)

    wa0, wg0 = W0[:, :_C], W0[:, _C:]
    wa1, wg1 = W1[:, :_C], W1[:, _C:]
    wa2, wg2 = W2[:, :_C], W2[:, _C:]
    wac = wa0.T
    wan = jnp.concatenate([(wa1 + wg1).T, wa2.T], axis=1)
    wgc = wg0.T
    wgn = jnp.concatenate([(-wg1).T, wg2.T], axis=1)
    bias = jnp.broadcast_to(
        jnp.concatenate([b0, b1, b2])[None, :], (8, _OC3))
    a_tab, g_tab = _proj(xc_p, xn_p, wac, wan, wgc, wgn, bias)

    h = _sc_gather(g_tab.reshape(_B * _N, _OC3),
                   idx.reshape(_NW, _NCHUNK, _CH))

    gb = jnp.stack([jnp.concatenate([g0, g1, g2]),
                    jnp.concatenate([be0, be1, be2])])
    st = _stats(a_tab, h.reshape(_B, _EPB, _OC3), gb)

    return _edge(a_tab, h.reshape(_B, _K, _N, _OC3), st)


# speedup vs baseline: 2.6681x; 1.0337x over previous
import functools

import jax
import jax.numpy as jnp
from jax import lax
from jax.experimental import pallas as pl
from jax.experimental.pallas import tpu as pltpu
from jax.experimental.pallas import tpu_sc as plsc

_B, _C, _N, _K = 4, 32, 4096, 24
_OC3 = 3 * _C
_E = _B * _N * _K
_EPB = _N * _K
_RB = 256
_MB = 2048
_NB = 256
_NW = 32
_CH = 128
_NCHUNK = _E // (_NW * _CH)


def _knn_body(xvt_ref, xv_ref, out_ref):
    b = pl.program_id(0)
    xr = xvt_ref[0]
    xv = xv_ref[0]
    dot = lax.dot_general(xr, xv, (((1,), (0,)), ((), ())),
                          preferred_element_type=jnp.float32)
    inner = -2.0 * dot
    xx_r = jnp.sum(xr * xr, axis=1, keepdims=True)
    xx_c = jnp.sum(xv * xv, axis=0, keepdims=True)
    s = -xx_r - inner - xx_c
    col = lax.broadcasted_iota(jnp.int32, (_RB, _N), 1).astype(jnp.float32)
    tcol = lax.broadcasted_iota(jnp.int32, (_RB, _K), 1).astype(jnp.float32)
    acc = jnp.zeros((_RB, _K), jnp.float32)
    big = jnp.float32(3.0e38)
    for t in range(_K):
        m = jnp.max(s, axis=1, keepdims=True)
        cand = jnp.where(s == m, col, big)
        am = jnp.min(cand, axis=1, keepdims=True)
        acc = jnp.where(tcol == t, jnp.broadcast_to(am, (_RB, _K)), acc)
        s = jnp.where(col == am, jnp.float32(-3.0e38), s)
    out_ref[0] = acc.astype(jnp.int32) + b * _N


def _knn(xvt, xv):
    return pl.pallas_call(
        _knn_body,
        grid=(_B, _N // _RB),
        in_specs=[
            pl.BlockSpec((1, _RB, _C), lambda b, r: (b, r, 0)),
            pl.BlockSpec((1, _C, _N), lambda b, r: (b, 0, 0)),
        ],
        out_specs=pl.BlockSpec((1, _RB, _K), lambda b, r: (b, r, 0)),
        out_shape=jax.ShapeDtypeStruct((_B, _N, _K), jnp.int32),
    )(xvt, xv)


def _proj_body(xc_ref, xn_ref, wac_ref, wan_ref, wgc_ref, wgn_ref, bias_ref,
               a_ref, g_ref):
    xc = xc_ref[0]
    xn = xn_ref[0]
    dn = lambda x, w: lax.dot_general(x, w, (((1,), (0,)), ((), ())),
                                      preferred_element_type=jnp.float32)
    a = jnp.concatenate([dn(xc, wac_ref[...]), dn(xn, wan_ref[...])], axis=1)
    a_ref[0] = a + bias_ref[0:1, :]
    g_ref[0] = jnp.concatenate([dn(xc, wgc_ref[...]), dn(xn, wgn_ref[...])],
                               axis=1)


def _proj(xc_p, xn_p, wac, wan, wgc, wgn, bias):
    full = lambda shape: pl.BlockSpec(shape, lambda b: (0,) * len(shape))
    return pl.pallas_call(
        _proj_body,
        grid=(_B,),
        in_specs=[
            pl.BlockSpec((1, _N, _C), lambda b: (b, 0, 0)),
            pl.BlockSpec((1, _N, _C), lambda b: (b, 0, 0)),
            full((_C, _C)), full((_C, 2 * _C)),
            full((_C, _C)), full((_C, 2 * _C)),
            full((8, _OC3)),
        ],
        out_specs=[
            pl.BlockSpec((1, _N, _OC3), lambda b: (b, 0, 0)),
            pl.BlockSpec((1, _N, _OC3), lambda b: (b, 0, 0)),
        ],
        out_shape=[
            jax.ShapeDtypeStruct((_B, _N, _OC3), jnp.float32),
            jax.ShapeDtypeStruct((_B, _N, _OC3), jnp.float32),
        ],
    )(xc_p, xn_p, wac, wan, wgc, wgn, bias)


def _gather_body(table_ref, idx_ref, out_ref, idx_v, rows_v,
                 sem_i, sem_g, sem_o):
    wid = lax.axis_index("s") * 2 + lax.axis_index("c")
    pltpu.async_copy(idx_ref.at[wid, 0], idx_v.at[0], sem_i)

    def pair(i, carry):
        for par in range(2):
            q = 2 * i + par
            obase = (wid * _NCHUNK + q) * _CH

            @pl.when(q >= 2)
            def _():
                pltpu.make_async_copy(
                    rows_v.at[par],
                    out_ref.at[pl.ds((wid * _NCHUNK + q - 2) * _CH, _CH)],
                    sem_o).wait()

            pltpu.make_async_copy(idx_ref.at[wid, q], idx_v.at[par],
                                  sem_i).wait()
            gat = pltpu.async_copy(table_ref.at[idx_v.at[par]],
                                   rows_v.at[par], sem_g)

            @pl.when(q + 1 < _NCHUNK)
            def _():
                pltpu.async_copy(idx_ref.at[wid, q + 1],
                                 idx_v.at[1 - par], sem_i)

            gat.wait()
            pltpu.async_copy(rows_v.at[par], out_ref.at[pl.ds(obase, _CH)],
                             sem_o)
        return carry

    lax.fori_loop(0, _NCHUNK // 2, pair, 0)
    for q in (_NCHUNK - 2, _NCHUNK - 1):
        pltpu.make_async_copy(
            rows_v.at[q % 2],
            out_ref.at[pl.ds((wid * _NCHUNK + q) * _CH, _CH)],
            sem_o).wait()


def _sc_gather(table, idx):
    mesh = plsc.VectorSubcoreMesh(core_axis_name="c", subcore_axis_name="s")
    f = functools.partial(
        pl.kernel,
        mesh=mesh,
        compiler_params=pltpu.CompilerParams(use_tc_tiling_on_sc=False),
        out_type=jax.ShapeDtypeStruct((_E, _OC3), jnp.float32),
        scratch_types=[
            pltpu.VMEM((2, _CH), jnp.int32),
            pltpu.VMEM((2, _CH, _OC3), jnp.float32),
            pltpu.SemaphoreType.DMA,
            pltpu.SemaphoreType.DMA,
            pltpu.SemaphoreType.DMA,
        ],
    )(_gather_body)
    return f(table, idx)


def _stats_body(a_ref, h_ref, gb_ref, o_ref):
    b = pl.program_id(0)
    pb = pl.program_id(1)
    y = a_ref[0] + h_ref[0]
    ps = jnp.sum(y, axis=0, keepdims=True)
    pq = jnp.sum(y * y, axis=0, keepdims=True)

    @pl.when((b == 0) & (pb == 0))
    def _():
        o_ref[...] = jnp.zeros((8, _OC3), jnp.float32)

    o_ref[0:1, :] += ps
    o_ref[1:2, :] += pq

    @pl.when((b == _B - 1) & (pb == _EPB // _MB - 1))
    def _():
        inv_e = jnp.float32(1.0 / _E)
        mean = o_ref[0:1, :] * inv_e
        var = o_ref[1:2, :] * inv_e - mean * mean
        scale = gb_ref[0:1, :] / jnp.sqrt(var + 1e-5)
        shift = gb_ref[1:2, :] - mean * scale
        o_ref[0:1, :] = scale
        o_ref[1:2, :] = shift


def _stats(a_tab, h_edges, gb):
    return pl.pallas_call(
        _stats_body,
        grid=(_B, _EPB // _MB),
        in_specs=[
            pl.BlockSpec((1, _MB, _OC3),
                         lambda b, p: (b, p % (_N // _MB), 0)),
            pl.BlockSpec((1, _MB, _OC3), lambda b, p: (b, p, 0)),
            pl.BlockSpec((2, _OC3), lambda b, p: (0, 0)),
        ],
        out_specs=pl.BlockSpec((8, _OC3), lambda b, p: (0, 0)),
        out_shape=jax.ShapeDtypeStruct((8, _OC3), jnp.float32),
    )(a_tab, h_edges, gb)


def _edge_body(a_ref, h_ref, st_ref, xc_ref, xn_ref):
    scale = st_ref[0:1, :]
    shift = st_ref[1:2, :]
    a = a_ref[0]
    zts = []
    for j in range(_K):
        y = a + h_ref[0, j]
        z = y * scale + shift
        z = jnp.where(z >= 0, z, 0.01 * z)
        zts.append(jnp.transpose(z, (1, 0)))
    z3 = jnp.stack(zts, axis=0)

    def group(br, t):
        u0 = 32 * t
        c_lo, j0 = u0 // _K, u0 % _K
        c0 = br * _C + c_lo
        p1 = z3[j0:_K, c0, :]
        p2 = z3[0:32 - (_K - j0), c0 + 1, :]
        return jnp.concatenate([p1, p2], axis=0)

    acc = jnp.zeros((_C, _NB), jnp.float32)
    mx = None
    for t in range(_K):
        g1 = group(1, t)
        e = jnp.exp(g1 - jnp.max(g1, axis=0, keepdims=True))
        attn = e / jnp.sum(e, axis=0, keepdims=True)
        acc = acc + attn * group(0, t)
        g2 = group(2, t)
        mx = g2 if t == 0 else jnp.maximum(mx, g2)
    xc_ref[0] = acc
    xn_ref[0] = mx


def _edge(a_tab, h4, st):
    return pl.pallas_call(
        _edge_body,
        grid=(_B, _N // _NB),
        in_specs=[
            pl.BlockSpec((1, _NB, _OC3), lambda b, n: (b, n, 0)),
            pl.BlockSpec((1, _K, _NB, _OC3), lambda b, n: (b, 0, n, 0)),
            pl.BlockSpec((8, _OC3), lambda b, n: (0, 0)),
        ],
        out_specs=[
            pl.BlockSpec((1, _C, _NB), lambda b, n: (b, 0, n)),
            pl.BlockSpec((1, _C, _NB), lambda b, n: (b, 0, n)),
        ],
        out_shape=[
            jax.ShapeDtypeStruct((_B, _C, _N), jnp.float32),
            jax.ShapeDtypeStruct((_B, _C, _N), jnp.float32),
        ],
    )(a_tab, h4, st)


def kernel(xc, xn, W0, b0, g0, be0, W1, b1, g1, be1, W2, b2, g2, be2):
    xc_p = jnp.transpose(xc, (0, 2, 1))
    xn_p = jnp.transpose(xn, (0, 2, 1))
    xv = xc_p.reshape(_B, _C, _N)
    xvt = jnp.transpose(xv, (0, 2, 1))

    idx = _knn(xvt, xv)

    wa0, wg0 = W0[:, :_C], W0[:, _C:]
    wa1, wg1 = W1[:, :_C], W1[:, _C:]
    wa2, wg2 = W2[:, :_C], W2[:, _C:]
    wac = wa0.T
    wan = jnp.concatenate([(wa1 + wg1).T, wa2.T], axis=1)
    wgc = wg0.T
    wgn = jnp.concatenate([(-wg1).T, wg2.T], axis=1)
    bias = jnp.broadcast_to(
        jnp.concatenate([b0, b1, b2])[None, :], (8, _OC3))
    a_tab, g_tab = _proj(xc_p, xn_p, wac, wan, wgc, wgn, bias)

    h = _sc_gather(g_tab.reshape(_B * _N, _OC3),
                   idx.reshape(_NW, _NCHUNK, _CH))

    gb = jnp.stack([jnp.concatenate([g0, g1, g2]),
                    jnp.concatenate([be0, be1, be2])])
    st = _stats(a_tab, h.reshape(_B, _EPB, _OC3), gb)

    return _edge(a_tab, h.reshape(_B, _K, _N, _OC3), st)
